# baseline (speedup 1.0000x reference)
import functools

import jax
import jax.numpy as jnp
from jax import lax
from jax.experimental import pallas as pl
from jax.experimental.pallas import tpu as pltpu
from jax.experimental.pallas import tpu_sc as plsc

N = 100000
E = 1600000
D = 32

NC = 2
NS = 16
L = 16
GPR = 128 // L

CH = 512
CHR = CH // 128
EP = 1605632
ROWS = EP // 128
NCH_EDGE = EP // (NC * NS) // CH
NCH_NODE = EP // NS // CH

NH = N // NC
ACC_H = 50048
ACC_N = 100096
TRASH_H = NH
TRASH_N = N
SL_H = 3128
SL_HT = NH - 15 * SL_H
SL_N = ACC_N // NS
SL_F = 6256
SL_FT = N - 15 * SL_F

_mesh = plsc.VectorSubcoreMesh(core_axis_name="c", subcore_axis_name="s")
_sc_params = pltpu.CompilerParams(use_tc_tiling_on_sc=False,
                                  needs_layout_passes=False)

_f32 = jnp.float32
_i32 = jnp.int32


def _idx2():
    return [pltpu.VMEM((CHR, 128), _i32) for _ in range(2)]


def _sem2():
    return [pltpu.SemaphoreType.DMA for _ in range(2)]


def _iota16():
    return lax.iota(_i32, L)


def _drain(sem, src_like, dst):
    pltpu.make_async_copy(src_like, dst, sem).wait()


def _off8(x):
    return pl.multiple_of(x, 8)


def _copy_half(src, src_off, dst, dst_off, s):
    @pl.when(s < NS - 1)
    def _():
        pltpu.sync_copy(src.at[pl.ds(_off8(src_off + s * SL_H), SL_H)],
                        dst.at[pl.ds(_off8(dst_off + s * SL_H), SL_H)])

    @pl.when(s == NS - 1)
    def _():
        pltpu.sync_copy(src.at[pl.ds(_off8(src_off + 15 * SL_H), SL_HT)],
                        dst.at[pl.ds(_off8(dst_off + 15 * SL_H), SL_HT)])


def _copy_full(src, src_off, dst, dst_off, s):
    @pl.when(s < NS - 1)
    def _():
        pltpu.sync_copy(src.at[pl.ds(_off8(src_off + s * SL_F), SL_F)],
                        dst.at[pl.ds(_off8(dst_off + s * SL_F), SL_F)])

    @pl.when(s == NS - 1)
    def _():
        pltpu.sync_copy(src.at[pl.ds(_off8(src_off + 15 * SL_F), SL_FT)],
                        dst.at[pl.ds(_off8(dst_off + 15 * SL_F), SL_FT)])


@functools.partial(
    pl.kernel,
    out_type=jax.ShapeDtypeStruct((NC, ACC_N, 16), _f32),
    mesh=_mesh,
    compiler_params=_sc_params,
    scratch_types=[pltpu.VMEM_SHARED((ACC_N, 16), _f32)] + _idx2() + _idx2()
    + [pltpu.VMEM((CH, 16), _f32)] + _sem2() + _sem2(),
)
def _sc_count(dst2d, z16, out, acc, didx0, didx1, tidx0, tidx1, ones_rows,
              isem0, isem1, ssem0, ssem1):
    didx = (didx0, didx1)
    tidx = (tidx0, tidx1)
    isem = (isem0, isem1)
    ssem = (ssem0, ssem1)
    c = lax.axis_index("c")
    s = lax.axis_index("s")
    w = c * NS + s
    pltpu.sync_copy(z16.at[pl.ds(s * SL_N, SL_N)], acc.at[pl.ds(s * SL_N, SL_N)])
    for col in range(16):
        colv = jnp.full((L,), col, _i32)
        val = jnp.full((L,), 1.0 if col == 0 else 0.0, _f32)
        for g in range(CH // L):
            plsc.store_scatter(ones_rows, [_iota16() + g * L, colv], val)
    plsc.subcore_barrier()

    def _r0(jj):
        return w * (NCH_EDGE * CHR) + jj * CHR

    pltpu.async_copy(dst2d.at[pl.ds(_r0(0), CHR)], didx[0], isem[0])
    pltpu.async_copy(dst2d.at[pl.ds(_r0(1), CHR)], didx[1], isem[1])

    @pl.loop(0, NCH_EDGE, step=2)
    def _chunk(j):
        for b in range(2):
            jj = j + b
            _drain(isem[b], dst2d.at[pl.ds(0, CHR)], didx[b])

            @pl.when(jj >= 2)
            def _():
                _drain(ssem[b], z16.at[pl.ds(0, CH)], ones_rows)

            ebase = _r0(jj) * 128
            for g in range(CH // L):
                rr, cc = g // GPR, g % GPR
                d = didx[b][rr, pl.ds(cc * L, L)]
                eid = _iota16() + (ebase + g * L)
                tidx[b][rr, pl.ds(cc * L, L)] = jnp.where(eid < E, d, TRASH_N)
            for rr in range(CHR):
                pltpu.async_copy(ones_rows.at[pl.ds(rr * 128, 128)],
                                 acc.at[tidx[b].at[rr]], ssem[b], add=True)

            @pl.when(jj + 2 < NCH_EDGE)
            def _():
                pltpu.async_copy(dst2d.at[pl.ds(_r0(jj + 2), CHR)],
                                 didx[b], isem[b])

    for b in range(2):
        _drain(ssem[b], z16.at[pl.ds(0, CH)], ones_rows)
    plsc.subcore_barrier()
    pltpu.sync_copy(acc.at[pl.ds(s * SL_N, SL_N)],
                    out.at[c, pl.ds(s * SL_N, SL_N)])


@functools.partial(
    pl.kernel,
    out_type=jax.ShapeDtypeStruct((NC, N, 16), _f32),
    mesh=_mesh,
    compiler_params=_sc_params,
    scratch_types=[pltpu.VMEM_SHARED((ACC_N, 16), _f32)]
    + _idx2() + _idx2() + _idx2()
    + [pltpu.VMEM((CH, 16), _f32) for _ in range(2)]
    + _sem2() + _sem2() + _sem2(),
)
def _sc_sage_agg(src2d, dst2d, x_st, z16, out, acc,
                 sidx0, sidx1, didx0, didx1, tidx0, tidx1, rows0, rows1,
                 isem0, isem1, gsem0, gsem1, ssem0, ssem1):
    sidx = (sidx0, sidx1)
    didx = (didx0, didx1)
    tidx = (tidx0, tidx1)
    rows = (rows0, rows1)
    isem = (isem0, isem1)
    gsem = (gsem0, gsem1)
    ssem = (ssem0, ssem1)
    c = lax.axis_index("c")
    s = lax.axis_index("s")
    tab_off = c * N
    pltpu.sync_copy(z16.at[pl.ds(s * SL_N, SL_N)], acc.at[pl.ds(s * SL_N, SL_N)])
    plsc.subcore_barrier()

    def _r0(jj):
        return s * (NCH_NODE * CHR) + jj * CHR

    def _issue_idx(jj, b, sem):
        pltpu.async_copy(src2d.at[pl.ds(_r0(jj), CHR)], sidx[b], sem)
        pltpu.async_copy(dst2d.at[pl.ds(_r0(jj), CHR)], didx[b], sem)

    def _issue_gather(b, sem):
        for g in range(CH // L):
            rr, cc = g // GPR, g % GPR
            sidx[b][rr, pl.ds(cc * L, L)] = (
                sidx[b][rr, pl.ds(cc * L, L)] + tab_off)
        for rr in range(CHR):
            pltpu.async_copy(x_st.at[sidx[b].at[rr]],
                             rows[b].at[pl.ds(rr * 128, 128)], sem)

    pltpu.sync_copy(src2d.at[pl.ds(_r0(0), CHR)], sidx[0])
    pltpu.sync_copy(dst2d.at[pl.ds(_r0(0), CHR)], didx[0])
    _issue_gather(0, gsem[0])
    _issue_idx(1, 1, isem[1])

    @pl.loop(0, NCH_NODE, step=2)
    def _chunk(j):
        for b in range(2):
            jj = j + b
            b2 = 1 - b

            @pl.when(jj + 1 < NCH_NODE)
            def _():
                _drain(isem[b2], src2d.at[pl.ds(0, CHR)], sidx[b2])
                _drain(isem[b2], src2d.at[pl.ds(0, CHR)], didx[b2])

                @pl.when(jj >= 1)
                def _():
                    _drain(ssem[b2], z16.at[pl.ds(0, CH)], rows[b2])

                _issue_gather(b2, gsem[b2])

            _drain(gsem[b], z16.at[pl.ds(0, CH)], rows[b])
            ebase = _r0(jj) * 128
            for g in range(CH // L):
                rr, cc = g // GPR, g % GPR
                d = didx[b][rr, pl.ds(cc * L, L)]
                eid = _iota16() + (ebase + g * L)
                tidx[b][rr, pl.ds(cc * L, L)] = jnp.where(eid < E, d, TRASH_N)
            for rr in range(CHR):
                pltpu.async_copy(rows[b].at[pl.ds(rr * 128, 128)],
                                 acc.at[tidx[b].at[rr]], ssem[b], add=True)

            @pl.when(jj + 2 < NCH_NODE)
            def _():
                _issue_idx(jj + 2, b, isem[b])

    for b in range(2):
        _drain(ssem[b], z16.at[pl.ds(0, CH)], rows[b])
    plsc.subcore_barrier()
    _copy_full(acc, 0, out.at[c], 0, s)


@functools.partial(
    pl.kernel,
    out_type=(jax.ShapeDtypeStruct((ROWS, 128), _f32),
              jax.ShapeDtypeStruct((ROWS, 128), _f32)),
    mesh=_mesh,
    compiler_params=_sc_params,
    scratch_types=_idx2() + _idx2()
    + [pltpu.VMEM((CH, 16), _f32) for _ in range(4)]
    + [pltpu.VMEM((CHR, 128), _f32) for _ in range(4)]
    + _sem2() + _sem2() + _sem2(),
)
def _sc_gat_pre(src2d, dst2d, a_tab, ex0_hbm, ex1_hbm,
                sidx0, sidx1, didx0, didx1, as0b, as1b, ad0b, ad1b,
                ex0b0, ex0b1, ex1b0, ex1b1,
                isem0, isem1, gsem0, gsem1, wsem0, wsem1):
    sidx = (sidx0, sidx1)
    didx = (didx0, didx1)
    arows_s = (as0b, as1b)
    arows_d = (ad0b, ad1b)
    ex0b = (ex0b0, ex0b1)
    ex1b = (ex1b0, ex1b1)
    isem = (isem0, isem1)
    gsem = (gsem0, gsem1)
    wsem = (wsem0, wsem1)
    c = lax.axis_index("c")
    s = lax.axis_index("s")
    w = c * NS + s

    def _r0(jj):
        return w * (NCH_EDGE * CHR) + jj * CHR

    def _issue_idx(jj, b, sem):
        pltpu.async_copy(src2d.at[pl.ds(_r0(jj), CHR)], sidx[b], sem)
        pltpu.async_copy(dst2d.at[pl.ds(_r0(jj), CHR)], didx[b], sem)

    def _issue_gather(b, sem):
        for rr in range(CHR):
            pltpu.async_copy(a_tab.at[sidx[b].at[rr]],
                             arows_s[b].at[pl.ds(rr * 128, 128)], sem)
            pltpu.async_copy(a_tab.at[didx[b].at[rr]],
                             arows_d[b].at[pl.ds(rr * 128, 128)], sem)

    pltpu.sync_copy(src2d.at[pl.ds(_r0(0), CHR)], sidx[0])
    pltpu.sync_copy(dst2d.at[pl.ds(_r0(0), CHR)], didx[0])
    _issue_gather(0, gsem[0])
    _issue_idx(1, 1, isem[1])

    @pl.loop(0, NCH_EDGE, step=2)
    def _chunk(j):
        for b in range(2):
            jj = j + b
            b2 = 1 - b

            @pl.when(jj + 1 < NCH_EDGE)
            def _():
                _drain(isem[b2], src2d.at[pl.ds(0, CHR)], sidx[b2])
                _drain(isem[b2], src2d.at[pl.ds(0, CHR)], didx[b2])
                _issue_gather(b2, gsem[b2])

            @pl.when(jj >= 2)
            def _():
                _drain(wsem[b], ex0_hbm.at[pl.ds(0, CHR)], ex0b[b])
                _drain(wsem[b], ex0_hbm.at[pl.ds(0, CHR)], ex1b[b])

            _drain(gsem[b], a_tab.at[pl.ds(0, CH)], arows_s[b])
            _drain(gsem[b], a_tab.at[pl.ds(0, CH)], arows_d[b])
            ebase = _r0(jj) * 128
            z = jnp.zeros((L,), _i32)
            for g in range(CH // L):
                rr, cc = g // GPR, g % GPR
                rid = _iota16() + g * L
                as_0 = plsc.load_gather(arows_s[b], [rid, z])
                as_1 = plsc.load_gather(arows_s[b], [rid, z + 1])
                ad_0 = plsc.load_gather(arows_d[b], [rid, z + 2])
                ad_1 = plsc.load_gather(arows_d[b], [rid, z + 3])
                t0 = as_0 + ad_0
                t1 = as_1 + ad_1
                e0 = jnp.exp(jnp.maximum(t0, t0 * 0.2))
                e1 = jnp.exp(jnp.maximum(t1, t1 * 0.2))
                eid = _iota16() + (ebase + g * L)
                pad_ok = eid < E
                ex0b[b][rr, pl.ds(cc * L, L)] = jnp.where(pad_ok, e0, 0.0)
                ex1b[b][rr, pl.ds(cc * L, L)] = jnp.where(pad_ok, e1, 0.0)
            pltpu.async_copy(ex0b[b], ex0_hbm.at[pl.ds(_r0(jj), CHR)], wsem[b])
            pltpu.async_copy(ex1b[b], ex1_hbm.at[pl.ds(_r0(jj), CHR)], wsem[b])

            @pl.when(jj + 2 < NCH_EDGE)
            def _():
                _issue_idx(jj + 2, b, isem[b])

    for b in range(2):
        _drain(wsem[b], ex0_hbm.at[pl.ds(0, CHR)], ex0b[b])
        _drain(wsem[b], ex0_hbm.at[pl.ds(0, CHR)], ex1b[b])


@functools.partial(
    pl.kernel,
    out_type=jax.ShapeDtypeStruct((NC, ACC_N, 16), _f32),
    mesh=_mesh,
    compiler_params=_sc_params,
    scratch_types=[pltpu.VMEM_SHARED((ACC_N, 16), _f32)] + _idx2() + _idx2()
    + [pltpu.VMEM((CHR, 128), _f32) for _ in range(4)]
    + [pltpu.VMEM((CH, 16), _f32) for _ in range(2)]
    + _sem2() + _sem2(),
)
def _sc_gat_den(dst2d, ex0_hbm, ex1_hbm, z16, out,
                acc, didx0, didx1, tidx0, tidx1,
                ex0b0, ex0b1, ex1b0, ex1b1, rows0, rows1,
                isem0, isem1, ssem0, ssem1):
    didx = (didx0, didx1)
    tidx = (tidx0, tidx1)
    ex0b = (ex0b0, ex0b1)
    ex1b = (ex1b0, ex1b1)
    rows = (rows0, rows1)
    isem = (isem0, isem1)
    ssem = (ssem0, ssem1)
    c = lax.axis_index("c")
    s = lax.axis_index("s")
    w = c * NS + s
    pltpu.sync_copy(z16.at[pl.ds(s * SL_N, SL_N)], acc.at[pl.ds(s * SL_N, SL_N)])
    for col in range(16):
        colv = jnp.full((L,), col, _i32)
        for bb in range(2):
            for g in range(CH // L):
                plsc.store_scatter(rows[bb], [_iota16() + g * L, colv],
                                   jnp.zeros((L,), _f32))
    plsc.subcore_barrier()

    def _r0(jj):
        return w * (NCH_EDGE * CHR) + jj * CHR

    def _issue_idx(jj, b, sem):
        pltpu.async_copy(dst2d.at[pl.ds(_r0(jj), CHR)], didx[b], sem)
        pltpu.async_copy(ex0_hbm.at[pl.ds(_r0(jj), CHR)], ex0b[b], sem)
        pltpu.async_copy(ex1_hbm.at[pl.ds(_r0(jj), CHR)], ex1b[b], sem)

    _issue_idx(0, 0, isem[0])
    _issue_idx(1, 1, isem[1])

    @pl.loop(0, NCH_EDGE, step=2)
    def _chunk(j):
        for b in range(2):
            jj = j + b
            _drain(isem[b], dst2d.at[pl.ds(0, CHR)], didx[b])
            _drain(isem[b], ex0_hbm.at[pl.ds(0, CHR)], ex0b[b])
            _drain(isem[b], ex0_hbm.at[pl.ds(0, CHR)], ex1b[b])

            @pl.when(jj >= 2)
            def _():
                _drain(ssem[b], z16.at[pl.ds(0, CH)], rows[b])

            z = jnp.zeros((L,), _i32)
            for g in range(CH // L):
                rr, cc = g // GPR, g % GPR
                rid = _iota16() + g * L
                e0 = ex0b[b][rr, pl.ds(cc * L, L)]
                e1 = ex1b[b][rr, pl.ds(cc * L, L)]
                plsc.store_scatter(rows[b], [rid, z], e0)
                plsc.store_scatter(rows[b], [rid, z + 1], e1)
                tidx[b][rr, pl.ds(cc * L, L)] = didx[b][rr, pl.ds(cc * L, L)]
            for rr in range(CHR):
                pltpu.async_copy(rows[b].at[pl.ds(rr * 128, 128)],
                                 acc.at[tidx[b].at[rr]], ssem[b], add=True)

            @pl.when(jj + 2 < NCH_EDGE)
            def _():
                _issue_idx(jj + 2, b, isem[b])

    for b in range(2):
        _drain(ssem[b], z16.at[pl.ds(0, CH)], rows[b])
    plsc.subcore_barrier()
    pltpu.sync_copy(acc.at[pl.ds(s * SL_N, SL_N)],
                    out.at[c, pl.ds(s * SL_N, SL_N)])


@functools.partial(
    pl.kernel,
    out_type=(jax.ShapeDtypeStruct((NC, N, 16), _f32),
              jax.ShapeDtypeStruct((NC, N, 16), _f32)),
    mesh=_mesh,
    compiler_params=_sc_params,
    scratch_types=[pltpu.VMEM_SHARED((ACC_N, 16), _f32)]
    + _idx2() + _idx2() + _idx2()
    + [pltpu.VMEM((CH, 16), _f32) for _ in range(2)]
    + [pltpu.VMEM((CHR, 128), _f32) for _ in range(2)]
    + _sem2() + _sem2() + _sem2(),
)
def _sc_gat_num(src2d, dst2d, xl0_flat, xl1_flat, ex0_hbm, ex1_hbm,
                n0i, n1i, z16, out0, out1,
                acc, sidx0, sidx1, didx0, didx1, tidx0, tidx1,
                rows0, rows1, exb0, exb1,
                isem0, isem1, gsem0, gsem1, ssem0, ssem1):
    sidx = (sidx0, sidx1)
    didx = (didx0, didx1)
    tidx = (tidx0, tidx1)
    rows = (rows0, rows1)
    exb = (exb0, exb1)
    isem = (isem0, isem1)
    gsem = (gsem0, gsem1)
    ssem = (ssem0, ssem1)
    c = lax.axis_index("c")
    s = lax.axis_index("s")
    tab_off = c * N

    def _r0(jj):
        return s * (NCH_NODE * CHR) + jj * CHR

    for xl_st, ex_hbm, init_hbm, out in ((xl0_flat, ex0_hbm, n0i, out0),
                                         (xl1_flat, ex1_hbm, n1i, out1)):
        _copy_full(init_hbm.at[c], 0, acc, 0, s)

        @pl.when(s == 0)
        def _():
            pltpu.sync_copy(z16.at[pl.ds(N, ACC_N - N)],
                            acc.at[pl.ds(N, ACC_N - N)])

        plsc.subcore_barrier()

        def _issue_idx(jj, b, sem, ex_hbm=ex_hbm):
            pltpu.async_copy(src2d.at[pl.ds(_r0(jj), CHR)], sidx[b], sem)
            pltpu.async_copy(dst2d.at[pl.ds(_r0(jj), CHR)], didx[b], sem)
            pltpu.async_copy(ex_hbm.at[pl.ds(_r0(jj), CHR)], exb[b], sem)

        def _issue_gather(b, sem, xl_st=xl_st):
            for g in range(CH // L):
                rr, cc = g // GPR, g % GPR
                sidx[b][rr, pl.ds(cc * L, L)] = (
                    sidx[b][rr, pl.ds(cc * L, L)] + tab_off)
            for rr in range(CHR):
                pltpu.async_copy(xl_st.at[sidx[b].at[rr]],
                                 rows[b].at[pl.ds(rr * 128, 128)], sem)

        pltpu.sync_copy(src2d.at[pl.ds(_r0(0), CHR)], sidx[0])
        pltpu.sync_copy(dst2d.at[pl.ds(_r0(0), CHR)], didx[0])
        pltpu.sync_copy(ex_hbm.at[pl.ds(_r0(0), CHR)], exb[0])
        _issue_gather(0, gsem[0])
        _issue_idx(1, 1, isem[1])

        @pl.loop(0, NCH_NODE, step=2)
        def _chunk(j, _issue_idx=_issue_idx, _issue_gather=_issue_gather,
                   ex_hbm=ex_hbm):
            for b in range(2):
                jj = j + b
                b2 = 1 - b

                @pl.when(jj + 1 < NCH_NODE)
                def _():
                    _drain(isem[b2], src2d.at[pl.ds(0, CHR)], sidx[b2])
                    _drain(isem[b2], src2d.at[pl.ds(0, CHR)], didx[b2])
                    _drain(isem[b2], ex_hbm.at[pl.ds(0, CHR)], exb[b2])

                    @pl.when(jj >= 1)
                    def _():
                        _drain(ssem[b2], z16.at[pl.ds(0, CH)], rows[b2])

                    _issue_gather(b2, gsem[b2])

                _drain(gsem[b], z16.at[pl.ds(0, CH)], rows[b])

                for g in range(CH // L):
                    rr, cc = g // GPR, g % GPR
                    rid = _iota16() + g * L
                    ex16 = exb[b][rr, pl.ds(cc * L, L)]
                    for dcol in range(16):
                        dv = jnp.full((L,), dcol, _i32)
                        v = plsc.load_gather(rows[b], [rid, dv])
                        plsc.store_scatter(rows[b], [rid, dv], v * ex16)

                for g in range(CH // L):
                    rr, cc = g // GPR, g % GPR
                    d = didx[b][rr, pl.ds(cc * L, L)]
                    eid = _iota16() + (_r0(jj) * 128 + g * L)
                    tidx[b][rr, pl.ds(cc * L, L)] = jnp.where(
                        eid < E, d, TRASH_N)
                for rr in range(CHR):
                    pltpu.async_copy(rows[b].at[pl.ds(rr * 128, 128)],
                                     acc.at[tidx[b].at[rr]], ssem[b],
                                     add=True)

                @pl.when(jj + 2 < NCH_NODE)
                def _():
                    _issue_idx(jj + 2, b, isem[b])

        for b in range(2):
            _drain(ssem[b], z16.at[pl.ds(0, CH)], rows[b])
        plsc.subcore_barrier()
        _copy_full(acc, 0, out.at[c], 0, s)
        plsc.subcore_barrier()


@functools.partial(
    pl.kernel,
    out_type=jax.ShapeDtypeStruct((EP, D), _f32),
    mesh=_mesh,
    compiler_params=_sc_params,
    scratch_types=_idx2() + _idx2()
    + [pltpu.VMEM((CH, D), _f32) for _ in range(4)]
    + _sem2() + _sem2() + _sem2(),
)
def _sc_mlp_edge(src2d, dst2d, p_tab, q_tab, out,
                 sidx0, sidx1, didx0, didx1, bufp0, bufp1, bufq0, bufq1,
                 isem0, isem1, gsem0, gsem1, wsem0, wsem1):
    sidx = (sidx0, sidx1)
    didx = (didx0, didx1)
    bufp = (bufp0, bufp1)
    bufq = (bufq0, bufq1)
    isem = (isem0, isem1)
    gsem = (gsem0, gsem1)
    wsem = (wsem0, wsem1)
    c = lax.axis_index("c")
    s = lax.axis_index("s")
    w = c * NS + s

    def _r0(jj):
        return w * (NCH_EDGE * CHR) + jj * CHR

    def _issue_idx(jj, b, sem):
        pltpu.async_copy(src2d.at[pl.ds(_r0(jj), CHR)], sidx[b], sem)
        pltpu.async_copy(dst2d.at[pl.ds(_r0(jj), CHR)], didx[b], sem)

    def _issue_gather(b, sem):
        for rr in range(CHR):
            pltpu.async_copy(p_tab.at[sidx[b].at[rr]],
                             bufp[b].at[pl.ds(rr * 128, 128)], sem)
            pltpu.async_copy(q_tab.at[didx[b].at[rr]],
                             bufq[b].at[pl.ds(rr * 128, 128)], sem)

    pltpu.sync_copy(src2d.at[pl.ds(_r0(0), CHR)], sidx[0])
    pltpu.sync_copy(dst2d.at[pl.ds(_r0(0), CHR)], didx[0])
    _issue_gather(0, gsem[0])
    _issue_idx(1, 1, isem[1])

    @pl.loop(0, NCH_EDGE, step=2)
    def _chunk(j):
        for b in range(2):
            jj = j + b
            b2 = 1 - b

            @pl.when(jj + 1 < NCH_EDGE)
            def _():
                _drain(isem[b2], src2d.at[pl.ds(0, CHR)], sidx[b2])
                _drain(isem[b2], src2d.at[pl.ds(0, CHR)], didx[b2])

                @pl.when(jj >= 1)
                def _():
                    _drain(wsem[b2], p_tab.at[pl.ds(0, CH)], bufp[b2])

                _issue_gather(b2, gsem[b2])

            _drain(gsem[b], p_tab.at[pl.ds(0, CH)], bufp[b])
            _drain(gsem[b], p_tab.at[pl.ds(0, CH)], bufq[b])

            for e in range(CH):
                a0 = bufp[b][e, pl.ds(0, L)] + bufq[b][e, pl.ds(0, L)]
                a1 = bufp[b][e, pl.ds(L, L)] + bufq[b][e, pl.ds(L, L)]
                bufp[b][e, pl.ds(0, L)] = a0
                bufp[b][e, pl.ds(L, L)] = a1

            pltpu.async_copy(bufp[b], out.at[pl.ds(_r0(jj) * 128, CH)],
                             wsem[b])

            @pl.when(jj + 2 < NCH_EDGE)
            def _():
                _issue_idx(jj + 2, b, isem[b])

    for b in range(2):
        _drain(wsem[b], p_tab.at[pl.ds(0, CH)], bufp[b])


_BN = 1000
_BE = 4096


def _mm(a, w):
    return jnp.dot(a, w)


def _nblk(shape):
    return pl.BlockSpec(shape, lambda i: (0,) * (len(shape) - 2) + (i, 0))


def _wblk(shape):
    nd = len(shape)
    return pl.BlockSpec(shape, lambda i, _nd=nd: (0,) * _nd)


def _split(res):
    return jnp.stack([res[:, :16], res[:, 16:]], axis=0)


def _joined(st):
    return jnp.concatenate([st[0], st[1]], axis=1)


def _tc_sage_body(agg, cntp, x, wl, bl, wr, o):
    cnt = cntp[0, :, 0:1] + cntp[1, :, 0:1]
    aggm = _joined(agg[...]) / jnp.maximum(cnt, 1.0)
    res = jax.nn.relu(
        _mm(aggm, wl[...]) + bl[...][None, :]
        + _mm(x[...], wr[...]))
    o[...] = _split(res)


def _tc_sage(agg, cntp, x, wl, bl, wr):
    return pl.pallas_call(
        _tc_sage_body,
        grid=(N // _BN,),
        in_specs=[_nblk((NC, _BN, 16)), _nblk((NC, _BN, 16)), _nblk((_BN, D)),
                  _wblk((D, D)), _wblk((D,)), _wblk((D, D))],
        out_specs=_nblk((NC, _BN, 16)),
        out_shape=jax.ShapeDtypeStruct((NC, N, 16), _f32),
    )(agg, cntp, x, wl, bl, wr)


def _tc_sage_gatprep_body(agg, cntp, x, wl, bl, wr, gw, asr, adr,
                          xl0o, xl1o, ao, exso, n0o, n1o):
    cnt = cntp[0, :, 0:1] + cntp[1, :, 0:1]
    aggm = _joined(agg[...]) / jnp.maximum(cnt, 1.0)
    x2 = jax.nn.relu(
        _mm(aggm, wl[...]) + bl[...][None, :]
        + _mm(_joined(x[...]), wr[...]))
    xl = _mm(x2, gw[...])
    xl0 = xl[:, :D]
    xl1 = xl[:, D:]
    a_s = asr[...]
    a_d = adr[...]
    as0 = jnp.dot(xl0, a_s[0][:, None])
    as1 = jnp.dot(xl1, a_s[1][:, None])
    ad0 = jnp.dot(xl0, a_d[0][:, None])
    ad1 = jnp.dot(xl1, a_d[1][:, None])
    t0 = as0 + ad0
    t1 = as1 + ad1
    exs0 = jnp.exp(jnp.maximum(t0, t0 * 0.2))
    exs1 = jnp.exp(jnp.maximum(t1, t1 * 0.2))
    z = jnp.zeros((x2.shape[0], 12), _f32)
    xl0o[...] = _split(xl0)
    xl1o[...] = _split(xl1)
    ao[...] = jnp.concatenate([as0, as1, ad0, ad1, z], axis=1)
    exso[...] = jnp.concatenate([exs0, exs1, z, z[:, :2]], axis=1)
    n0o[...] = _split(xl0 * exs0)
    n1o[...] = _split(xl1 * exs1)


def _tc_sage_gatprep(agg, cntp, x, wl, bl, wr, gw, att_src, att_dst):
    outs = (
        jax.ShapeDtypeStruct((NC, N, 16), _f32),
        jax.ShapeDtypeStruct((NC, N, 16), _f32),
        jax.ShapeDtypeStruct((N, 16), _f32),
        jax.ShapeDtypeStruct((N, 16), _f32),
        jax.ShapeDtypeStruct((NC, N, 16), _f32),
        jax.ShapeDtypeStruct((NC, N, 16), _f32),
    )
    return pl.pallas_call(
        _tc_sage_gatprep_body,
        grid=(N // _BN,),
        in_specs=[_nblk((NC, _BN, 16)), _nblk((NC, _BN, 16)),
                  _nblk((NC, _BN, 16)),
                  _wblk((D, D)), _wblk((D,)), _wblk((D, D)),
                  _wblk((D, 2 * D)), _wblk((2, D)), _wblk((2, D))],
        out_specs=(_nblk((NC, _BN, 16)), _nblk((NC, _BN, 16)),
                   _nblk((_BN, 16)), _nblk((_BN, 16)),
                   _nblk((NC, _BN, 16)), _nblk((NC, _BN, 16))),
        out_shape=outs,
    )(agg, cntp, x, wl, bl, wr, gw, att_src, att_dst)


def _tc_gat_fin_body(n0, n1, denp, exs, gb, w1ab, b1, po, qo):
    d0 = exs[:, 0:1] + denp[0, :, 0:1] + denp[1, :, 0:1] + 1e-16
    d1 = exs[:, 1:2] + denp[0, :, 1:2] + denp[1, :, 1:2] + 1e-16
    x3 = jax.nn.relu(
        (_joined(n0[...]) / d0 + _joined(n1[...]) / d1) * 0.5
        + gb[...][None, :])
    po[...] = _mm(x3, w1ab[...][:D])
    qo[...] = _mm(x3, w1ab[...][D:]) + b1[...][None, :]


def _tc_gat_fin(n0, n1, denp, exs, gb, w1ab, b1):
    return pl.pallas_call(
        _tc_gat_fin_body,
        grid=(N // _BN,),
        in_specs=[_nblk((NC, _BN, 16)), _nblk((NC, _BN, 16)),
                  _nblk((NC, _BN, 16)),
                  _nblk((_BN, 16)), _wblk((D,)), _wblk((2 * D, D)),
                  _wblk((D,))],
        out_specs=(_nblk((_BN, D)), _nblk((_BN, D))),
        out_shape=(jax.ShapeDtypeStruct((N, D), _f32),
                   jax.ShapeDtypeStruct((N, D), _f32)),
    )(n0, n1, denp, exs, gb, w1ab, b1)


def _tc_final_body(h, ea, w1c, w2, o):
    g = jax.nn.relu(h[...] + _mm(ea[...], w1c[...]))
    o[...] = jnp.sum(g * w2[...][None, :], axis=1).reshape(_BE // 128, 128)


def _tc_final(h, ea_p, w1c, w2row):
    out2d = pl.pallas_call(
        _tc_final_body,
        grid=(EP // _BE,),
        in_specs=[_nblk((_BE, D)), _nblk((_BE, 16)), _wblk((16, D)),
                  _wblk((D,))],
        out_specs=_nblk((_BE // 128, 128)),
        out_shape=jax.ShapeDtypeStruct((EP // 128, 128), _f32),
    )(h, ea_p, w1c, w2row)
    return out2d.reshape(EP)[:E]


def kernel(edge_index, edge_attr, node_emb,
           sage1_Wl, sage1_bl, sage1_Wr,
           sage2_Wl, sage2_bl, sage2_Wr,
           gat_W, gat_att_src, gat_att_dst, gat_bias,
           mlp_W1, mlp_b1, mlp_W2, mlp_b2):
    src = edge_index[0]
    dst = edge_index[1]
    pad = EP - E
    src2d = jnp.concatenate([src, jnp.zeros((pad,), _i32)]).reshape(ROWS, 128)
    dst2d = jnp.concatenate([dst, jnp.zeros((pad,), _i32)]).reshape(ROWS, 128)
    z16 = jnp.zeros((ACC_N, 16), _f32)

    cntp = _sc_count(dst2d, z16)
    x_st = jnp.stack([node_emb[:, :16], node_emb[:, 16:]], axis=0)
    agg1 = _sc_sage_agg(src2d, dst2d, x_st.reshape(2 * N, 16), z16)
    x1 = _tc_sage(agg1, cntp, node_emb, sage1_Wl, sage1_bl, sage1_Wr)
    agg2 = _sc_sage_agg(src2d, dst2d, x1.reshape(2 * N, 16), z16)
    xl0, xl1, a_tab, exs, n0i, n1i = _tc_sage_gatprep(
        agg2, cntp, x1, sage2_Wl, sage2_bl, sage2_Wr,
        gat_W, gat_att_src, gat_att_dst)
    ex0, ex1 = _sc_gat_pre(src2d, dst2d, a_tab)
    denp = _sc_gat_den(dst2d, ex0, ex1, z16)
    num0, num1 = _sc_gat_num(src2d, dst2d, xl0.reshape(2 * N, 16),
                             xl1.reshape(2 * N, 16), ex0, ex1, n0i, n1i, z16)
    p_tab, q_tab = _tc_gat_fin(num0, num1, denp, exs, gat_bias,
                               mlp_W1[:2 * D], mlp_b1)
    h = _sc_mlp_edge(src2d, dst2d, p_tab, q_tab)
    ea_p = jnp.concatenate([edge_attr, jnp.zeros((pad, 16), _f32)])
    out = _tc_final(h, ea_p, mlp_W1[2 * D:], mlp_W2[:, 0])
    return out + mlp_b2[0]


# speedup vs baseline: 1.0456x; 1.0456x over previous
import functools

import jax
import jax.numpy as jnp
from jax import lax
from jax.experimental import pallas as pl
from jax.experimental.pallas import tpu as pltpu
from jax.experimental.pallas import tpu_sc as plsc

N = 100000
E = 1600000
D = 32

NC = 2
NS = 16
L = 16
GPR = 128 // L

CH = 512
CHR = CH // 128
EP = 1605632
ROWS = EP // 128
NCH_EDGE = EP // (NC * NS) // CH
NCH_NODE = EP // NS // CH

NH = N // NC
ACC_H = 50048
ACC_N = 100096
TRASH_H = NH
TRASH_N = N
SL_H = 3128
SL_HT = NH - 15 * SL_H
SL_N = ACC_N // NS
SL_F = 6256
SL_FT = N - 15 * SL_F

_mesh = plsc.VectorSubcoreMesh(core_axis_name="c", subcore_axis_name="s")
_sc_params = pltpu.CompilerParams(use_tc_tiling_on_sc=False,
                                  needs_layout_passes=False)

_f32 = jnp.float32
_i32 = jnp.int32


def _idx2():
    return [pltpu.VMEM((CHR, 128), _i32) for _ in range(2)]


def _sem2():
    return [pltpu.SemaphoreType.DMA for _ in range(2)]


def _iota16():
    return lax.iota(_i32, L)


def _drain(sem, src_like, dst):
    pltpu.make_async_copy(src_like, dst, sem).wait()


def _off8(x):
    return pl.multiple_of(x, 8)


def _copy_half(src, src_off, dst, dst_off, s):
    @pl.when(s < NS - 1)
    def _():
        pltpu.sync_copy(src.at[pl.ds(_off8(src_off + s * SL_H), SL_H)],
                        dst.at[pl.ds(_off8(dst_off + s * SL_H), SL_H)])

    @pl.when(s == NS - 1)
    def _():
        pltpu.sync_copy(src.at[pl.ds(_off8(src_off + 15 * SL_H), SL_HT)],
                        dst.at[pl.ds(_off8(dst_off + 15 * SL_H), SL_HT)])


def _copy_full(src, src_off, dst, dst_off, s):
    @pl.when(s < NS - 1)
    def _():
        pltpu.sync_copy(src.at[pl.ds(_off8(src_off + s * SL_F), SL_F)],
                        dst.at[pl.ds(_off8(dst_off + s * SL_F), SL_F)])

    @pl.when(s == NS - 1)
    def _():
        pltpu.sync_copy(src.at[pl.ds(_off8(src_off + 15 * SL_F), SL_FT)],
                        dst.at[pl.ds(_off8(dst_off + 15 * SL_F), SL_FT)])


@functools.partial(
    pl.kernel,
    out_type=jax.ShapeDtypeStruct((NC, ACC_N, 16), _f32),
    mesh=_mesh,
    compiler_params=_sc_params,
    scratch_types=[pltpu.VMEM_SHARED((ACC_N, 16), _f32)] + _idx2() + _idx2()
    + [pltpu.VMEM((CH, 16), _f32)] + _sem2() + _sem2(),
)
def _sc_count(dst2d, z16, out, acc, didx0, didx1, tidx0, tidx1, ones_rows,
              isem0, isem1, ssem0, ssem1):
    didx = (didx0, didx1)
    tidx = (tidx0, tidx1)
    isem = (isem0, isem1)
    ssem = (ssem0, ssem1)
    c = lax.axis_index("c")
    s = lax.axis_index("s")
    w = c * NS + s
    pltpu.sync_copy(z16.at[pl.ds(s * SL_N, SL_N)], acc.at[pl.ds(s * SL_N, SL_N)])
    for col in range(16):
        colv = jnp.full((L,), col, _i32)
        val = jnp.full((L,), 1.0 if col == 0 else 0.0, _f32)
        for g in range(CH // L):
            plsc.store_scatter(ones_rows, [_iota16() + g * L, colv], val)
    plsc.subcore_barrier()

    def _r0(jj):
        return w * (NCH_EDGE * CHR) + jj * CHR

    pltpu.async_copy(dst2d.at[pl.ds(_r0(0), CHR)], didx[0], isem[0])
    pltpu.async_copy(dst2d.at[pl.ds(_r0(1), CHR)], didx[1], isem[1])

    @pl.loop(0, NCH_EDGE, step=2)
    def _chunk(j):
        for b in range(2):
            jj = j + b
            _drain(isem[b], dst2d.at[pl.ds(0, CHR)], didx[b])

            @pl.when(jj >= 2)
            def _():
                _drain(ssem[b], z16.at[pl.ds(0, CH)], ones_rows)

            ebase = _r0(jj) * 128
            for g in range(CH // L):
                rr, cc = g // GPR, g % GPR
                d = didx[b][rr, pl.ds(cc * L, L)]
                eid = _iota16() + (ebase + g * L)
                tidx[b][rr, pl.ds(cc * L, L)] = jnp.where(eid < E, d, TRASH_N)
            for rr in range(CHR):
                pltpu.async_copy(ones_rows.at[pl.ds(rr * 128, 128)],
                                 acc.at[tidx[b].at[rr]], ssem[b], add=True)

            @pl.when(jj + 2 < NCH_EDGE)
            def _():
                pltpu.async_copy(dst2d.at[pl.ds(_r0(jj + 2), CHR)],
                                 didx[b], isem[b])

    for b in range(2):
        _drain(ssem[b], z16.at[pl.ds(0, CH)], ones_rows)
    plsc.subcore_barrier()
    pltpu.sync_copy(acc.at[pl.ds(s * SL_N, SL_N)],
                    out.at[c, pl.ds(s * SL_N, SL_N)])


@functools.partial(
    pl.kernel,
    out_type=jax.ShapeDtypeStruct((NC, N, 16), _f32),
    mesh=_mesh,
    compiler_params=_sc_params,
    scratch_types=[pltpu.VMEM_SHARED((ACC_N, 16), _f32)]
    + _idx2() + _idx2() + _idx2()
    + [pltpu.VMEM((CH, 16), _f32) for _ in range(2)]
    + _sem2() + _sem2() + _sem2(),
)
def _sc_sage_agg(src2d, dst2d, x_st, z16, out, acc,
                 sidx0, sidx1, didx0, didx1, tidx0, tidx1, rows0, rows1,
                 isem0, isem1, gsem0, gsem1, ssem0, ssem1):
    sidx = (sidx0, sidx1)
    didx = (didx0, didx1)
    tidx = (tidx0, tidx1)
    rows = (rows0, rows1)
    isem = (isem0, isem1)
    gsem = (gsem0, gsem1)
    ssem = (ssem0, ssem1)
    c = lax.axis_index("c")
    s = lax.axis_index("s")
    tab_off = c * N
    pltpu.sync_copy(z16.at[pl.ds(s * SL_N, SL_N)], acc.at[pl.ds(s * SL_N, SL_N)])
    plsc.subcore_barrier()

    def _r0(jj):
        return s * (NCH_NODE * CHR) + jj * CHR

    def _issue_idx(jj, b, sem):
        pltpu.async_copy(src2d.at[pl.ds(_r0(jj), CHR)], sidx[b], sem)
        pltpu.async_copy(dst2d.at[pl.ds(_r0(jj), CHR)], didx[b], sem)

    def _issue_gather(b, sem):
        for g in range(CH // L):
            rr, cc = g // GPR, g % GPR
            sidx[b][rr, pl.ds(cc * L, L)] = (
                sidx[b][rr, pl.ds(cc * L, L)] + tab_off)
        for rr in range(CHR):
            pltpu.async_copy(x_st.at[sidx[b].at[rr]],
                             rows[b].at[pl.ds(rr * 128, 128)], sem)

    pltpu.sync_copy(src2d.at[pl.ds(_r0(0), CHR)], sidx[0])
    pltpu.sync_copy(dst2d.at[pl.ds(_r0(0), CHR)], didx[0])
    _issue_gather(0, gsem[0])
    _issue_idx(1, 1, isem[1])

    @pl.loop(0, NCH_NODE, step=2)
    def _chunk(j):
        for b in range(2):
            jj = j + b
            b2 = 1 - b

            @pl.when(jj + 1 < NCH_NODE)
            def _():
                _drain(isem[b2], src2d.at[pl.ds(0, CHR)], sidx[b2])
                _drain(isem[b2], src2d.at[pl.ds(0, CHR)], didx[b2])

                @pl.when(jj >= 1)
                def _():
                    _drain(ssem[b2], z16.at[pl.ds(0, CH)], rows[b2])

                _issue_gather(b2, gsem[b2])

            _drain(gsem[b], z16.at[pl.ds(0, CH)], rows[b])
            ebase = _r0(jj) * 128
            for g in range(CH // L):
                rr, cc = g // GPR, g % GPR
                d = didx[b][rr, pl.ds(cc * L, L)]
                eid = _iota16() + (ebase + g * L)
                tidx[b][rr, pl.ds(cc * L, L)] = jnp.where(eid < E, d, TRASH_N)
            for rr in range(CHR):
                pltpu.async_copy(rows[b].at[pl.ds(rr * 128, 128)],
                                 acc.at[tidx[b].at[rr]], ssem[b], add=True)

            @pl.when(jj + 2 < NCH_NODE)
            def _():
                _issue_idx(jj + 2, b, isem[b])

    for b in range(2):
        _drain(ssem[b], z16.at[pl.ds(0, CH)], rows[b])
    plsc.subcore_barrier()
    _copy_full(acc, 0, out.at[c], 0, s)


@functools.partial(
    pl.kernel,
    out_type=(jax.ShapeDtypeStruct((ROWS, 128), _f32),
              jax.ShapeDtypeStruct((ROWS, 128), _f32)),
    mesh=_mesh,
    compiler_params=_sc_params,
    scratch_types=_idx2() + _idx2()
    + [pltpu.VMEM((CH, 16), _f32) for _ in range(4)]
    + [pltpu.VMEM((CHR, 128), _f32) for _ in range(4)]
    + _sem2() + _sem2() + _sem2(),
)
def _sc_gat_pre(src2d, dst2d, a_tab, ex0_hbm, ex1_hbm,
                sidx0, sidx1, didx0, didx1, as0b, as1b, ad0b, ad1b,
                ex0b0, ex0b1, ex1b0, ex1b1,
                isem0, isem1, gsem0, gsem1, wsem0, wsem1):
    sidx = (sidx0, sidx1)
    didx = (didx0, didx1)
    arows_s = (as0b, as1b)
    arows_d = (ad0b, ad1b)
    ex0b = (ex0b0, ex0b1)
    ex1b = (ex1b0, ex1b1)
    isem = (isem0, isem1)
    gsem = (gsem0, gsem1)
    wsem = (wsem0, wsem1)
    c = lax.axis_index("c")
    s = lax.axis_index("s")
    w = c * NS + s

    def _r0(jj):
        return w * (NCH_EDGE * CHR) + jj * CHR

    def _issue_idx(jj, b, sem):
        pltpu.async_copy(src2d.at[pl.ds(_r0(jj), CHR)], sidx[b], sem)
        pltpu.async_copy(dst2d.at[pl.ds(_r0(jj), CHR)], didx[b], sem)

    def _issue_gather(b, sem):
        for rr in range(CHR):
            pltpu.async_copy(a_tab.at[sidx[b].at[rr]],
                             arows_s[b].at[pl.ds(rr * 128, 128)], sem)
            pltpu.async_copy(a_tab.at[didx[b].at[rr]],
                             arows_d[b].at[pl.ds(rr * 128, 128)], sem)

    pltpu.sync_copy(src2d.at[pl.ds(_r0(0), CHR)], sidx[0])
    pltpu.sync_copy(dst2d.at[pl.ds(_r0(0), CHR)], didx[0])
    _issue_gather(0, gsem[0])
    _issue_idx(1, 1, isem[1])

    @pl.loop(0, NCH_EDGE, step=2)
    def _chunk(j):
        for b in range(2):
            jj = j + b
            b2 = 1 - b

            @pl.when(jj + 1 < NCH_EDGE)
            def _():
                _drain(isem[b2], src2d.at[pl.ds(0, CHR)], sidx[b2])
                _drain(isem[b2], src2d.at[pl.ds(0, CHR)], didx[b2])
                _issue_gather(b2, gsem[b2])

            @pl.when(jj >= 2)
            def _():
                _drain(wsem[b], ex0_hbm.at[pl.ds(0, CHR)], ex0b[b])
                _drain(wsem[b], ex0_hbm.at[pl.ds(0, CHR)], ex1b[b])

            _drain(gsem[b], a_tab.at[pl.ds(0, CH)], arows_s[b])
            _drain(gsem[b], a_tab.at[pl.ds(0, CH)], arows_d[b])
            ebase = _r0(jj) * 128
            z = jnp.zeros((L,), _i32)
            for g in range(CH // L):
                rr, cc = g // GPR, g % GPR
                rid = _iota16() + g * L
                as_0 = plsc.load_gather(arows_s[b], [rid, z])
                as_1 = plsc.load_gather(arows_s[b], [rid, z + 1])
                ad_0 = plsc.load_gather(arows_d[b], [rid, z + 2])
                ad_1 = plsc.load_gather(arows_d[b], [rid, z + 3])
                t0 = as_0 + ad_0
                t1 = as_1 + ad_1
                e0 = jnp.exp(jnp.maximum(t0, t0 * 0.2))
                e1 = jnp.exp(jnp.maximum(t1, t1 * 0.2))
                eid = _iota16() + (ebase + g * L)
                pad_ok = eid < E
                ex0b[b][rr, pl.ds(cc * L, L)] = jnp.where(pad_ok, e0, 0.0)
                ex1b[b][rr, pl.ds(cc * L, L)] = jnp.where(pad_ok, e1, 0.0)
            pltpu.async_copy(ex0b[b], ex0_hbm.at[pl.ds(_r0(jj), CHR)], wsem[b])
            pltpu.async_copy(ex1b[b], ex1_hbm.at[pl.ds(_r0(jj), CHR)], wsem[b])

            @pl.when(jj + 2 < NCH_EDGE)
            def _():
                _issue_idx(jj + 2, b, isem[b])

    for b in range(2):
        _drain(wsem[b], ex0_hbm.at[pl.ds(0, CHR)], ex0b[b])
        _drain(wsem[b], ex0_hbm.at[pl.ds(0, CHR)], ex1b[b])


@functools.partial(
    pl.kernel,
    out_type=jax.ShapeDtypeStruct((NC, ACC_N, 16), _f32),
    mesh=_mesh,
    compiler_params=_sc_params,
    scratch_types=[pltpu.VMEM_SHARED((ACC_N, 16), _f32)] + _idx2() + _idx2()
    + [pltpu.VMEM((CHR, 128), _f32) for _ in range(4)]
    + [pltpu.VMEM((CH, 16), _f32) for _ in range(2)]
    + _sem2() + _sem2(),
)
def _sc_gat_den(dst2d, ex0_hbm, ex1_hbm, z16, out,
                acc, didx0, didx1, tidx0, tidx1,
                ex0b0, ex0b1, ex1b0, ex1b1, rows0, rows1,
                isem0, isem1, ssem0, ssem1):
    didx = (didx0, didx1)
    tidx = (tidx0, tidx1)
    ex0b = (ex0b0, ex0b1)
    ex1b = (ex1b0, ex1b1)
    rows = (rows0, rows1)
    isem = (isem0, isem1)
    ssem = (ssem0, ssem1)
    c = lax.axis_index("c")
    s = lax.axis_index("s")
    w = c * NS + s
    pltpu.sync_copy(z16.at[pl.ds(s * SL_N, SL_N)], acc.at[pl.ds(s * SL_N, SL_N)])
    for col in range(16):
        colv = jnp.full((L,), col, _i32)
        for bb in range(2):
            for g in range(CH // L):
                plsc.store_scatter(rows[bb], [_iota16() + g * L, colv],
                                   jnp.zeros((L,), _f32))
    plsc.subcore_barrier()

    def _r0(jj):
        return w * (NCH_EDGE * CHR) + jj * CHR

    def _issue_idx(jj, b, sem):
        pltpu.async_copy(dst2d.at[pl.ds(_r0(jj), CHR)], didx[b], sem)
        pltpu.async_copy(ex0_hbm.at[pl.ds(_r0(jj), CHR)], ex0b[b], sem)
        pltpu.async_copy(ex1_hbm.at[pl.ds(_r0(jj), CHR)], ex1b[b], sem)

    _issue_idx(0, 0, isem[0])
    _issue_idx(1, 1, isem[1])

    @pl.loop(0, NCH_EDGE, step=2)
    def _chunk(j):
        for b in range(2):
            jj = j + b
            _drain(isem[b], dst2d.at[pl.ds(0, CHR)], didx[b])
            _drain(isem[b], ex0_hbm.at[pl.ds(0, CHR)], ex0b[b])
            _drain(isem[b], ex0_hbm.at[pl.ds(0, CHR)], ex1b[b])

            @pl.when(jj >= 2)
            def _():
                _drain(ssem[b], z16.at[pl.ds(0, CH)], rows[b])

            z = jnp.zeros((L,), _i32)
            for g in range(CH // L):
                rr, cc = g // GPR, g % GPR
                rid = _iota16() + g * L
                e0 = ex0b[b][rr, pl.ds(cc * L, L)]
                e1 = ex1b[b][rr, pl.ds(cc * L, L)]
                plsc.store_scatter(rows[b], [rid, z], e0)
                plsc.store_scatter(rows[b], [rid, z + 1], e1)
                tidx[b][rr, pl.ds(cc * L, L)] = didx[b][rr, pl.ds(cc * L, L)]
            for rr in range(CHR):
                pltpu.async_copy(rows[b].at[pl.ds(rr * 128, 128)],
                                 acc.at[tidx[b].at[rr]], ssem[b], add=True)

            @pl.when(jj + 2 < NCH_EDGE)
            def _():
                _issue_idx(jj + 2, b, isem[b])

    for b in range(2):
        _drain(ssem[b], z16.at[pl.ds(0, CH)], rows[b])
    plsc.subcore_barrier()
    pltpu.sync_copy(acc.at[pl.ds(s * SL_N, SL_N)],
                    out.at[c, pl.ds(s * SL_N, SL_N)])


@functools.partial(
    pl.kernel,
    out_type=jax.ShapeDtypeStruct((NC, N, 16), _f32),
    mesh=_mesh,
    compiler_params=_sc_params,
    scratch_types=[pltpu.VMEM_SHARED((ACC_N, 16), _f32)]
    + _idx2() + _idx2() + _idx2()
    + [pltpu.VMEM((CH, 16), _f32) for _ in range(2)]
    + [pltpu.VMEM((CHR, 128), _f32) for _ in range(2)]
    + _sem2() + _sem2() + _sem2(),
)
def _sc_gat_num(src2d, dst2d, xl_st, ex_hbm, init_hbm, z16, out,
                acc, sidx0, sidx1, didx0, didx1, tidx0, tidx1,
                rows0, rows1, exb0, exb1,
                isem0, isem1, gsem0, gsem1, ssem0, ssem1):
    sidx = (sidx0, sidx1)
    didx = (didx0, didx1)
    tidx = (tidx0, tidx1)
    rows = (rows0, rows1)
    exb = (exb0, exb1)
    isem = (isem0, isem1)
    gsem = (gsem0, gsem1)
    ssem = (ssem0, ssem1)
    c = lax.axis_index("c")
    s = lax.axis_index("s")
    tab_off = c * N
    _copy_full(init_hbm.at[c], 0, acc, 0, s)

    @pl.when(s == 0)
    def _():
        pltpu.sync_copy(z16.at[pl.ds(N, ACC_N - N)],
                        acc.at[pl.ds(N, ACC_N - N)])

    plsc.subcore_barrier()

    def _r0(jj):
        return s * (NCH_NODE * CHR) + jj * CHR

    def _issue_idx(jj, b, sem):
        pltpu.async_copy(src2d.at[pl.ds(_r0(jj), CHR)], sidx[b], sem)
        pltpu.async_copy(dst2d.at[pl.ds(_r0(jj), CHR)], didx[b], sem)
        pltpu.async_copy(ex_hbm.at[pl.ds(_r0(jj), CHR)], exb[b], sem)

    def _issue_gather(b, sem):
        for g in range(CH // L):
            rr, cc = g // GPR, g % GPR
            sidx[b][rr, pl.ds(cc * L, L)] = (
                sidx[b][rr, pl.ds(cc * L, L)] + tab_off)
        for rr in range(CHR):
            pltpu.async_copy(xl_st.at[sidx[b].at[rr]],
                             rows[b].at[pl.ds(rr * 128, 128)], sem)

    pltpu.sync_copy(src2d.at[pl.ds(_r0(0), CHR)], sidx[0])
    pltpu.sync_copy(dst2d.at[pl.ds(_r0(0), CHR)], didx[0])
    pltpu.sync_copy(ex_hbm.at[pl.ds(_r0(0), CHR)], exb[0])
    _issue_gather(0, gsem[0])
    _issue_idx(1, 1, isem[1])

    @pl.loop(0, NCH_NODE, step=2)
    def _chunk(j):
        for b in range(2):
            jj = j + b
            b2 = 1 - b

            @pl.when(jj + 1 < NCH_NODE)
            def _():
                _drain(isem[b2], src2d.at[pl.ds(0, CHR)], sidx[b2])
                _drain(isem[b2], src2d.at[pl.ds(0, CHR)], didx[b2])
                _drain(isem[b2], ex_hbm.at[pl.ds(0, CHR)], exb[b2])

                @pl.when(jj >= 1)
                def _():
                    _drain(ssem[b2], z16.at[pl.ds(0, CH)], rows[b2])

                _issue_gather(b2, gsem[b2])

            _drain(gsem[b], z16.at[pl.ds(0, CH)], rows[b])

            for g in range(CH // L):
                rr, cc = g // GPR, g % GPR
                rid = _iota16() + g * L
                ex16 = exb[b][rr, pl.ds(cc * L, L)]
                for dcol in range(16):
                    dv = jnp.full((L,), dcol, _i32)
                    v = plsc.load_gather(rows[b], [rid, dv])
                    plsc.store_scatter(rows[b], [rid, dv], v * ex16)

            for g in range(CH // L):
                rr, cc = g // GPR, g % GPR
                d = didx[b][rr, pl.ds(cc * L, L)]
                eid = _iota16() + (_r0(jj) * 128 + g * L)
                tidx[b][rr, pl.ds(cc * L, L)] = jnp.where(eid < E, d, TRASH_N)
            for rr in range(CHR):
                pltpu.async_copy(rows[b].at[pl.ds(rr * 128, 128)],
                                 acc.at[tidx[b].at[rr]], ssem[b], add=True)

            @pl.when(jj + 2 < NCH_NODE)
            def _():
                _issue_idx(jj + 2, b, isem[b])

    for b in range(2):
        _drain(ssem[b], z16.at[pl.ds(0, CH)], rows[b])
    plsc.subcore_barrier()
    _copy_full(acc, 0, out.at[c], 0, s)


@functools.partial(
    pl.kernel,
    out_type=jax.ShapeDtypeStruct((EP, D), _f32),
    mesh=_mesh,
    compiler_params=_sc_params,
    scratch_types=_idx2() + _idx2()
    + [pltpu.VMEM((CH, D), _f32) for _ in range(4)]
    + _sem2() + _sem2() + _sem2(),
)
def _sc_mlp_edge(src2d, dst2d, p_tab, q_tab, out,
                 sidx0, sidx1, didx0, didx1, bufp0, bufp1, bufq0, bufq1,
                 isem0, isem1, gsem0, gsem1, wsem0, wsem1):
    sidx = (sidx0, sidx1)
    didx = (didx0, didx1)
    bufp = (bufp0, bufp1)
    bufq = (bufq0, bufq1)
    isem = (isem0, isem1)
    gsem = (gsem0, gsem1)
    wsem = (wsem0, wsem1)
    c = lax.axis_index("c")
    s = lax.axis_index("s")
    w = c * NS + s

    def _r0(jj):
        return w * (NCH_EDGE * CHR) + jj * CHR

    def _issue_idx(jj, b, sem):
        pltpu.async_copy(src2d.at[pl.ds(_r0(jj), CHR)], sidx[b], sem)
        pltpu.async_copy(dst2d.at[pl.ds(_r0(jj), CHR)], didx[b], sem)

    def _issue_gather(b, sem):
        for rr in range(CHR):
            pltpu.async_copy(p_tab.at[sidx[b].at[rr]],
                             bufp[b].at[pl.ds(rr * 128, 128)], sem)
            pltpu.async_copy(q_tab.at[didx[b].at[rr]],
                             bufq[b].at[pl.ds(rr * 128, 128)], sem)

    pltpu.sync_copy(src2d.at[pl.ds(_r0(0), CHR)], sidx[0])
    pltpu.sync_copy(dst2d.at[pl.ds(_r0(0), CHR)], didx[0])
    _issue_gather(0, gsem[0])
    _issue_idx(1, 1, isem[1])

    @pl.loop(0, NCH_EDGE, step=2)
    def _chunk(j):
        for b in range(2):
            jj = j + b
            b2 = 1 - b

            @pl.when(jj + 1 < NCH_EDGE)
            def _():
                _drain(isem[b2], src2d.at[pl.ds(0, CHR)], sidx[b2])
                _drain(isem[b2], src2d.at[pl.ds(0, CHR)], didx[b2])

                @pl.when(jj >= 1)
                def _():
                    _drain(wsem[b2], p_tab.at[pl.ds(0, CH)], bufp[b2])

                _issue_gather(b2, gsem[b2])

            _drain(gsem[b], p_tab.at[pl.ds(0, CH)], bufp[b])
            _drain(gsem[b], p_tab.at[pl.ds(0, CH)], bufq[b])

            for e in range(CH):
                a0 = bufp[b][e, pl.ds(0, L)] + bufq[b][e, pl.ds(0, L)]
                a1 = bufp[b][e, pl.ds(L, L)] + bufq[b][e, pl.ds(L, L)]
                bufp[b][e, pl.ds(0, L)] = a0
                bufp[b][e, pl.ds(L, L)] = a1

            pltpu.async_copy(bufp[b], out.at[pl.ds(_r0(jj) * 128, CH)],
                             wsem[b])

            @pl.when(jj + 2 < NCH_EDGE)
            def _():
                _issue_idx(jj + 2, b, isem[b])

    for b in range(2):
        _drain(wsem[b], p_tab.at[pl.ds(0, CH)], bufp[b])


_BN = 1000
_BE = 4096


def _mm(a, w):
    return jnp.dot(a, w)


def _nblk(shape):
    return pl.BlockSpec(shape, lambda i: (0,) * (len(shape) - 2) + (i, 0))


def _wblk(shape):
    nd = len(shape)
    return pl.BlockSpec(shape, lambda i, _nd=nd: (0,) * _nd)


def _split(res):
    return jnp.stack([res[:, :16], res[:, 16:]], axis=0)


def _joined(st):
    return jnp.concatenate([st[0], st[1]], axis=1)


def _tc_sage_body(agg, cntp, x, wl, bl, wr, o):
    cnt = cntp[0, :, 0:1] + cntp[1, :, 0:1]
    aggm = _joined(agg[...]) / jnp.maximum(cnt, 1.0)
    res = jax.nn.relu(
        _mm(aggm, wl[...]) + bl[...][None, :]
        + _mm(x[...], wr[...]))
    o[...] = _split(res)


def _tc_sage(agg, cntp, x, wl, bl, wr):
    return pl.pallas_call(
        _tc_sage_body,
        grid=(N // _BN,),
        in_specs=[_nblk((NC, _BN, 16)), _nblk((NC, _BN, 16)), _nblk((_BN, D)),
                  _wblk((D, D)), _wblk((D,)), _wblk((D, D))],
        out_specs=_nblk((NC, _BN, 16)),
        out_shape=jax.ShapeDtypeStruct((NC, N, 16), _f32),
    )(agg, cntp, x, wl, bl, wr)


def _tc_sage_gatprep_body(agg, cntp, x, wl, bl, wr, gw, asr, adr,
                          xl0o, xl1o, ao, exso, n0o, n1o):
    cnt = cntp[0, :, 0:1] + cntp[1, :, 0:1]
    aggm = _joined(agg[...]) / jnp.maximum(cnt, 1.0)
    x2 = jax.nn.relu(
        _mm(aggm, wl[...]) + bl[...][None, :]
        + _mm(_joined(x[...]), wr[...]))
    xl = _mm(x2, gw[...])
    xl0 = xl[:, :D]
    xl1 = xl[:, D:]
    a_s = asr[...]
    a_d = adr[...]
    as0 = jnp.dot(xl0, a_s[0][:, None])
    as1 = jnp.dot(xl1, a_s[1][:, None])
    ad0 = jnp.dot(xl0, a_d[0][:, None])
    ad1 = jnp.dot(xl1, a_d[1][:, None])
    t0 = as0 + ad0
    t1 = as1 + ad1
    exs0 = jnp.exp(jnp.maximum(t0, t0 * 0.2))
    exs1 = jnp.exp(jnp.maximum(t1, t1 * 0.2))
    z = jnp.zeros((x2.shape[0], 12), _f32)
    xl0o[...] = _split(xl0)
    xl1o[...] = _split(xl1)
    ao[...] = jnp.concatenate([as0, as1, ad0, ad1, z], axis=1)
    exso[...] = jnp.concatenate([exs0, exs1, z, z[:, :2]], axis=1)
    n0o[...] = _split(xl0 * exs0)
    n1o[...] = _split(xl1 * exs1)


def _tc_sage_gatprep(agg, cntp, x, wl, bl, wr, gw, att_src, att_dst):
    outs = (
        jax.ShapeDtypeStruct((NC, N, 16), _f32),
        jax.ShapeDtypeStruct((NC, N, 16), _f32),
        jax.ShapeDtypeStruct((N, 16), _f32),
        jax.ShapeDtypeStruct((N, 16), _f32),
        jax.ShapeDtypeStruct((NC, N, 16), _f32),
        jax.ShapeDtypeStruct((NC, N, 16), _f32),
    )
    return pl.pallas_call(
        _tc_sage_gatprep_body,
        grid=(N // _BN,),
        in_specs=[_nblk((NC, _BN, 16)), _nblk((NC, _BN, 16)),
                  _nblk((NC, _BN, 16)),
                  _wblk((D, D)), _wblk((D,)), _wblk((D, D)),
                  _wblk((D, 2 * D)), _wblk((2, D)), _wblk((2, D))],
        out_specs=(_nblk((NC, _BN, 16)), _nblk((NC, _BN, 16)),
                   _nblk((_BN, 16)), _nblk((_BN, 16)),
                   _nblk((NC, _BN, 16)), _nblk((NC, _BN, 16))),
        out_shape=outs,
    )(agg, cntp, x, wl, bl, wr, gw, att_src, att_dst)


def _tc_gat_fin_body(n0, n1, denp, exs, gb, w1ab, b1, po, qo):
    d0 = exs[:, 0:1] + denp[0, :, 0:1] + denp[1, :, 0:1] + 1e-16
    d1 = exs[:, 1:2] + denp[0, :, 1:2] + denp[1, :, 1:2] + 1e-16
    x3 = jax.nn.relu(
        (_joined(n0[...]) / d0 + _joined(n1[...]) / d1) * 0.5
        + gb[...][None, :])
    po[...] = _mm(x3, w1ab[...][:D])
    qo[...] = _mm(x3, w1ab[...][D:]) + b1[...][None, :]


def _tc_gat_fin(n0, n1, denp, exs, gb, w1ab, b1):
    return pl.pallas_call(
        _tc_gat_fin_body,
        grid=(N // _BN,),
        in_specs=[_nblk((NC, _BN, 16)), _nblk((NC, _BN, 16)),
                  _nblk((NC, _BN, 16)),
                  _nblk((_BN, 16)), _wblk((D,)), _wblk((2 * D, D)),
                  _wblk((D,))],
        out_specs=(_nblk((_BN, D)), _nblk((_BN, D))),
        out_shape=(jax.ShapeDtypeStruct((N, D), _f32),
                   jax.ShapeDtypeStruct((N, D), _f32)),
    )(n0, n1, denp, exs, gb, w1ab, b1)


def _tc_final_body(h, ea, w1c, w2, o):
    g = jax.nn.relu(h[...] + _mm(ea[...], w1c[...]))
    o[...] = jnp.sum(g * w2[...][None, :], axis=1).reshape(_BE // 128, 128)


def _tc_final(h, ea_p, w1c, w2row):
    out2d = pl.pallas_call(
        _tc_final_body,
        grid=(EP // _BE,),
        in_specs=[_nblk((_BE, D)), _nblk((_BE, 16)), _wblk((16, D)),
                  _wblk((D,))],
        out_specs=_nblk((_BE // 128, 128)),
        out_shape=jax.ShapeDtypeStruct((EP // 128, 128), _f32),
    )(h, ea_p, w1c, w2row)
    return out2d.reshape(EP)[:E]


def kernel(edge_index, edge_attr, node_emb,
           sage1_Wl, sage1_bl, sage1_Wr,
           sage2_Wl, sage2_bl, sage2_Wr,
           gat_W, gat_att_src, gat_att_dst, gat_bias,
           mlp_W1, mlp_b1, mlp_W2, mlp_b2):
    src = edge_index[0]
    dst = edge_index[1]
    pad = EP - E
    src2d = jnp.concatenate([src, jnp.zeros((pad,), _i32)]).reshape(ROWS, 128)
    dst2d = jnp.concatenate([dst, jnp.zeros((pad,), _i32)]).reshape(ROWS, 128)
    z16 = jnp.zeros((ACC_N, 16), _f32)

    cntp = _sc_count(dst2d, z16)
    x_st = jnp.stack([node_emb[:, :16], node_emb[:, 16:]], axis=0)
    agg1 = _sc_sage_agg(src2d, dst2d, x_st.reshape(2 * N, 16), z16)
    x1 = _tc_sage(agg1, cntp, node_emb, sage1_Wl, sage1_bl, sage1_Wr)
    agg2 = _sc_sage_agg(src2d, dst2d, x1.reshape(2 * N, 16), z16)
    xl0, xl1, a_tab, exs, n0i, n1i = _tc_sage_gatprep(
        agg2, cntp, x1, sage2_Wl, sage2_bl, sage2_Wr,
        gat_W, gat_att_src, gat_att_dst)
    ex0, ex1 = _sc_gat_pre(src2d, dst2d, a_tab)
    denp = _sc_gat_den(dst2d, ex0, ex1, z16)
    num0 = _sc_gat_num(src2d, dst2d, xl0.reshape(2 * N, 16), ex0, n0i, z16)
    num1 = _sc_gat_num(src2d, dst2d, xl1.reshape(2 * N, 16), ex1, n1i, z16)
    p_tab, q_tab = _tc_gat_fin(num0, num1, denp, exs, gat_bias,
                               mlp_W1[:2 * D], mlp_b1)
    h = _sc_mlp_edge(src2d, dst2d, p_tab, q_tab)
    ea_p = jnp.concatenate([edge_attr, jnp.zeros((pad, 16), _f32)])
    out = _tc_final(h, ea_p, mlp_W1[2 * D:], mlp_W2[:, 0])
    return out + mlp_b2[0]
